# baseline (speedup 1.0000x reference)
import numpy as np
import jax
import jax.numpy as jnp
from jax.experimental import pallas as pl

_STRIDES = (8, 16, 32)
_SIZES = (160, 80, 40)
_BATCH = 4
_NCLS = 80
_NMS_THR = 0.65
_SCORE_THR = 0.01
_NMS_PRE = 1000
_MAX_PER_IMG = 100
_S_LVL = tuple(s * s for s in _SIZES)
_S_TOT = sum(_S_LVL)
_BASES = (0, _S_LVL[0], _S_LVL[0] + _S_LVL[1])
_PAD = 1024
_NCHUNK = 33
_S_PADDED = _NCHUNK * _PAD


def _make_points():
    xs, ys = [], []
    for s, st in zip(_SIZES, _STRIDES):
        idx = np.arange(s * s)
        xs.append((idx % s).astype(np.float32) * st)
        ys.append((idx // s).astype(np.float32) * st)
    return np.stack([np.concatenate(xs), np.concatenate(ys)])[None]


_POINTS = _make_points()


def _score_decode_kernel(pts_ref, c0, r0, o0, c1, r1, o1, c2, r2, o2,
                         sc_ref, lab_ref, box_ref):
    cls_refs = (c0, c1, c2)
    reg_refs = (r0, r1, r2)
    obj_refs = (o0, o1, o2)
    for li in range(3):
        S = _S_LVL[li]
        st = float(_STRIDES[li])
        base = _BASES[li]
        cls = cls_refs[li][0]
        reg = reg_refs[li][0]
        obj = obj_refs[li][0]
        cmax = jnp.max(cls, axis=0, keepdims=True)
        iota = jax.lax.broadcasted_iota(jnp.int32, (_NCLS, S), 0)
        lab = jnp.min(jnp.where(cls == cmax, iota, 2 ** 30),
                      axis=0, keepdims=True)
        score = jax.nn.sigmoid(cmax) * jax.nn.sigmoid(obj)
        px = pts_ref[0, 0:1, base:base + S]
        py = pts_ref[0, 1:2, base:base + S]
        xc = reg[0:1, :] * st + px
        yc = reg[1:2, :] * st + py
        w = jnp.exp(reg[2:3, :]) * st
        h = jnp.exp(reg[3:4, :]) * st
        x1 = xc - w / 2.0
        y1 = yc - h / 2.0
        x2 = xc + w / 2.0
        y2 = yc + h / 2.0
        sc_ref[0, 0:1, base:base + S] = score
        lab_ref[0, 0:1, base:base + S] = lab
        box_ref[0, :, base:base + S] = jnp.concatenate([x1, y1, x2, y2], axis=0)


def _exchange(v, ix, pos, d, desc):
    if d < 128:
        axis, dd = v.ndim - 1, d
    else:
        axis, dd = v.ndim - 2, d // 128
    bit = (pos & d) != 0
    pv = jnp.where(bit, jnp.roll(v, dd, axis), jnp.roll(v, -dd, axis))
    pi = jnp.where(bit, jnp.roll(ix, dd, axis), jnp.roll(ix, -dd, axis))
    self_max = (v > pv) | ((v == pv) & (ix < pi))
    take_max = desc ^ bit
    keep_self = take_max == self_max
    return jnp.where(keep_self, v, pv), jnp.where(keep_self, ix, pi)


def _rev_chunk(x, pos):
    for k in range(10):
        d = 1 << k
        if d < 128:
            axis, dd = x.ndim - 1, d
        else:
            axis, dd = x.ndim - 2, d // 128
        bit = (pos & d) != 0
        x = jnp.where(bit, jnp.roll(x, dd, axis), jnp.roll(x, -dd, axis))
    return x


def _topk_kernel(sc_ref, val_ref, idx_ref, nmsval_ref):
    v = sc_ref[...]
    shp = v.shape
    pos = (jax.lax.broadcasted_iota(jnp.int32, shp, 2) * 128
           + jax.lax.broadcasted_iota(jnp.int32, shp, 3))
    gidx = jax.lax.broadcasted_iota(jnp.int32, shp, 1) * _PAD + pos
    v = jnp.where(v >= _SCORE_THR, v, -1.0)

    ix = gidx
    for k in range(1, 11):
        K = 1 << k
        desc = (pos & K) == 0
        for j in range(k - 1, -1, -1):
            v, ix = _exchange(v, ix, pos, 1 << j, desc)

    n = _NCHUNK
    while n > 1:
        m = (n + 1) // 2
        a_v, a_i = v[:, :n - m], ix[:, :n - m]
        b_v = _rev_chunk(v[:, m:n], pos[:, :n - m])
        b_i = _rev_chunk(ix[:, m:n], pos[:, :n - m])
        a_max = (a_v > b_v) | ((a_v == b_v) & (a_i < b_i))
        c_v = jnp.where(a_max, a_v, b_v)
        c_i = jnp.where(a_max, a_i, b_i)
        p2 = pos[:, :n - m]
        for j in range(9, -1, -1):
            c_v, c_i = _exchange(c_v, c_i, p2, 1 << j, True)
        if m > n - m:
            v = jnp.concatenate([c_v, v[:, n - m:m]], axis=1)
            ix = jnp.concatenate([c_i, ix[:, n - m:m]], axis=1)
        else:
            v, ix = c_v, c_i
        n = m

    top_v = v[:, 0]
    top_i = ix[:, 0]
    rank = pos[:, 0]
    val_ref[...] = top_v
    idx_ref[...] = top_i
    nmsval_ref[...] = jnp.where(rank < _NMS_PRE, top_v, -1.0)


def _nms_kernel(vals_ref, box_ref, lab_ref, out_ref):
    B = _BATCH
    vals = vals_ref[...]
    labf = lab_ref[...].astype(jnp.float32)
    boxes = box_ref[...]
    maxc = jnp.max(jnp.max(jnp.max(boxes, axis=3), axis=2), axis=1)
    off = labf * (maxc[:, None, None] + 1.0)
    bx1 = boxes[:, 0] + off
    by1 = boxes[:, 1] + off
    bx2 = boxes[:, 2] + off
    by2 = boxes[:, 3] + off
    area = (bx2 - bx1) * (by2 - by1)
    ar = (jax.lax.broadcasted_iota(jnp.int32, (B, 8, 128), 1) * 128
          + jax.lax.broadcasted_iota(jnp.int32, (B, 8, 128), 2))
    keep0 = jnp.where(vals > 0.0, 1.0, 0.0)
    NEG = -3.0e38

    def _bmax(x):
        return jnp.max(jnp.max(x, axis=2, keepdims=True), axis=1,
                       keepdims=True)

    def body(i, keep):
        sel = ar == i
        xi1 = _bmax(jnp.where(sel, bx1, NEG))
        yi1 = _bmax(jnp.where(sel, by1, NEG))
        xi2 = _bmax(jnp.where(sel, bx2, NEG))
        yi2 = _bmax(jnp.where(sel, by2, NEG))
        ki = _bmax(jnp.where(sel, keep, 0.0))
        w = jnp.maximum(jnp.minimum(xi2, bx2) - jnp.maximum(xi1, bx1), 0.0)
        h = jnp.maximum(jnp.minimum(yi2, by2) - jnp.maximum(yi1, by1), 0.0)
        inter = w * h
        ai = (xi2 - xi1) * (yi2 - yi1)
        iou = inter / (ai + area - inter + 1e-6)
        sup = jnp.where((iou > _NMS_THR) & (ar > i), ki, 0.0)
        return keep * (1.0 - sup)

    keep = jax.lax.fori_loop(0, _NMS_PRE, body, keep0)
    out_ref[...] = jnp.where(keep > 0.5, vals, -1.0)


def kernel(cls_out_0, reg_out_0, obj_out_0, cls_out_1, reg_out_1, obj_out_1,
           cls_out_2, reg_out_2, obj_out_2, images_hw):
    del images_hw
    B = _BATCH
    cls_l = [c.reshape(B, _NCLS, s * s)
             for c, s in zip((cls_out_0, cls_out_1, cls_out_2), _SIZES)]
    reg_l = [r.reshape(B, 4, s * s)
             for r, s in zip((reg_out_0, reg_out_1, reg_out_2), _SIZES)]
    obj_l = [o.reshape(B, 1, s * s)
             for o, s in zip((obj_out_0, obj_out_1, obj_out_2), _SIZES)]
    pts = jnp.asarray(_POINTS)

    in_specs = [pl.BlockSpec((1, 2, _S_TOT), lambda b: (0, 0, 0))]
    args = [pts]
    for li in range(3):
        S = _S_LVL[li]
        in_specs += [
            pl.BlockSpec((1, _NCLS, S), lambda b: (b, 0, 0)),
            pl.BlockSpec((1, 4, S), lambda b: (b, 0, 0)),
            pl.BlockSpec((1, 1, S), lambda b: (b, 0, 0)),
        ]
        args += [cls_l[li], reg_l[li], obj_l[li]]

    sc, lab, box = pl.pallas_call(
        _score_decode_kernel,
        grid=(B,),
        in_specs=in_specs,
        out_specs=[
            pl.BlockSpec((1, 1, _S_TOT), lambda b: (b, 0, 0)),
            pl.BlockSpec((1, 1, _S_TOT), lambda b: (b, 0, 0)),
            pl.BlockSpec((1, 4, _S_TOT), lambda b: (b, 0, 0)),
        ],
        out_shape=[
            jax.ShapeDtypeStruct((B, 1, _S_TOT), jnp.float32),
            jax.ShapeDtypeStruct((B, 1, _S_TOT), jnp.int32),
            jax.ShapeDtypeStruct((B, 4, _S_TOT), jnp.float32),
        ],
    )(*args)

    scores = sc[:, 0]
    scp = jnp.pad(scores, ((0, 0), (0, _S_PADDED - _S_TOT)),
                  constant_values=-1.0).reshape(B, _NCHUNK, 8, 128)
    vals_s, idx_s, nmsval = pl.pallas_call(
        _topk_kernel,
        out_shape=[
            jax.ShapeDtypeStruct((B, 8, 128), jnp.float32),
            jax.ShapeDtypeStruct((B, 8, 128), jnp.int32),
            jax.ShapeDtypeStruct((B, 8, 128), jnp.float32),
        ],
    )(scp)

    idx = idx_s.reshape(B, _PAD)
    idx_c = jnp.minimum(idx, _S_TOT - 1)
    rank = jnp.arange(_PAD)[None, :]
    bt = jnp.take_along_axis(box, idx_c[:, None, :], axis=2)
    lt = jnp.take_along_axis(lab[:, 0], idx_c, axis=1)
    boxp = jnp.where((rank < _NMS_PRE)[:, None, :], bt,
                     -1e30).reshape(B, 4, 8, 128)
    labp = jnp.where(rank < _NMS_PRE, lt, 0).reshape(B, 8, 128)

    fv = pl.pallas_call(
        _nms_kernel,
        out_shape=jax.ShapeDtypeStruct((B, 8, 128), jnp.float32),
    )(nmsval, boxp, labp)

    final = fv.reshape(B, _PAD)[:, :_NMS_PRE]
    fvals, fidx = jax.lax.top_k(final, _MAX_PER_IMG)
    out_boxes = jnp.take_along_axis(
        bt, fidx[:, None, :], axis=2).transpose(0, 2, 1)
    out_scores = jnp.maximum(fvals, 0.0)
    out_labels = jnp.take_along_axis(lt, fidx, axis=1)
    return out_boxes, out_scores, out_labels
